# baseline (speedup 1.0000x reference)
import functools

import jax
import jax.numpy as jnp
from jax import lax
from jax.experimental import pallas as pl
from jax.experimental.pallas import tpu as pltpu
from jax.experimental.pallas import tpu_sc as plsc

NUM_EMBEDDINGS = 1000000
EMBEDDING_DIM = 64
BATCH = 16384
NBLOCKS = 7813
SLABW = 384
NSLAB = 82
MAXBASE = 999552
NSTAGE = 128
SCAN_UNROLL = 16
UNROLL = 4


def _make_lookup():
    info = plsc.get_sparse_core_info()

    mesh = plsc.VectorSubcoreMesh(core_axis_name="c", subcore_axis_name="s")

    @functools.partial(
        pl.kernel,
        mesh=mesh,
        out_type=jax.ShapeDtypeStruct((BATCH * EMBEDDING_DIM,), jnp.float32),
        scratch_types=[
            pltpu.VMEM((BATCH,), jnp.int32),
            pltpu.VMEM((BATCH + 64,), jnp.int32),
            pltpu.VMEM((3, EMBEDDING_DIM, SLABW), jnp.float32),
            pltpu.VMEM((NSTAGE * EMBEDDING_DIM,), jnp.float32),
            pltpu.VMEM((EMBEDDING_DIM, 64), jnp.float32),
            pltpu.VMEM((EMBEDDING_DIM,), jnp.float32),
        ]
        + [pltpu.SemaphoreType.DMA] * 5,
        compiler_params=pltpu.CompilerParams(needs_layout_passes=False),
    )
    def lookup(x_hbm, tt_hbm, out_hbm, xall, lists, slabs, stage,
               fbtail, fbcol, sem_s0, sem_s1, sem_s2, sem_w, sem_fb):
        wid = lax.axis_index("s") * info.num_cores + lax.axis_index("c")
        c0 = (NBLOCKS * wid) >> 5
        liota = lax.iota(jnp.int32, 16)
        dvecs = [liota + 16 * q for q in range(4)]
        ssems = [sem_s0, sem_s1, sem_s2]

        def splat(x):
            return jnp.full((16,), x, dtype=jnp.int32)

        c0v = splat(c0)
        c1v = splat((NBLOCKS * (wid + 1)) >> 5)

        def slab_base(st):
            return jnp.minimum((c0 + 3 * st) * 128, MAXBASE)

        def start_slab(st, par):
            pltpu.async_copy(
                tt_hbm.at[:, pl.ds(pl.multiple_of(slab_base(st), 128), SLABW)],
                slabs.at[par], ssems[par],
            )

        def wait_slab(par):
            pltpu.make_async_copy(
                tt_hbm.at[:, pl.ds(0, SLABW)], slabs.at[par], ssems[par]
            ).wait()

        pltpu.sync_copy(x_hbm, xall)
        for par in range(3):
            start_slab(jnp.int32(par), par)

        def tail_fallback(pos, l):
            pltpu.async_copy(
                tt_hbm.at[:, pl.ds(999936, 64)], fbtail, sem_fb
            ).wait()
            lv = splat(l)
            for q in range(4):
                fbcol[pl.ds(q * 16, 16)] = plsc.load_gather(
                    fbtail, [dvecs[q], lv]
                )
            pltpu.sync_copy(fbcol, out_hbm.at[pl.ds(pos * 64, 64)])

        def scan_group(g, ptr):
            recs = []
            for u in range(SCAN_UNROLL):
                k = g * SCAN_UNROLL + u
                v = xall[pl.ds(k * 16, 16)]
                c = lax.shift_right_logical(v, 7)
                m = (c >= c0v) & (c < c1v)
                entryv = (
                    (k * 16 + liota) | ((v & 127) << 14) | ((c - c0v) << 21)
                )
                cnt = plsc.all_reduce_population_count(m)[0]
                recs.append((m, entryv, cnt))
            for m, entryv, cnt in recs:
                plsc.store_compressed(
                    lists.at[pl.ds(ptr, 16)], entryv, mask=m
                )
                ptr = ptr + cnt
            return ptr

        nlist = lax.fori_loop(
            0, (BATCH // 16) // SCAN_UNROLL, scan_group, jnp.int32(0)
        )
        nlv = splat(nlist)

        def drain_all():
            pltpu.make_async_copy(
                out_hbm.at[pl.ds(0, NSTAGE * 64)], stage, sem_w
            ).wait()

        def drain_n(n):
            def one(_, __):
                pltpu.make_async_copy(
                    out_hbm.at[pl.ds(0, 64)], stage.at[pl.ds(0, 64)], sem_w
                ).wait()
                return 0

            lax.fori_loop(0, n, one, 0)

        def process_slab(st, par, ms0):
            wait_slab(par)
            base = slab_base(st)
            stv = splat(st)
            three = splat(3)

            def rescan_one(k, ms):
                ev = lists[pl.ds(k * 16, 16)]
                valid = (k * 16 + liota) < nlv
                rbv = lax.shift_right_logical(ev, 21) & 255
                m = (rbv // three == stv) & valid
                cnt = plsc.all_reduce_population_count(m)[0]

                def pop_one(_, carry):
                    mask_i, ms_i = carry
                    mask = mask_i != 0
                    j = plsc.all_reduce_ffs(mask)[0]
                    jv = splat(j)
                    e = jnp.sum(jnp.where(liota == jv, ev, 0))
                    pos = e & 16383
                    l = lax.shift_right_logical(e, 14) & 127
                    rb = lax.shift_right_logical(e, 21) & 255
                    col = (c0 + rb) * 128 + l - base

                    @pl.when(col >= SLABW)
                    def _():
                        tail_fallback(pos, l)

                    @pl.when(col < SLABW)
                    def _():
                        slot = ms_i & (NSTAGE - 1)

                        @pl.when((slot == 0) & (ms_i > 0))
                        def _():
                            drain_all()

                        cv = splat(col)
                        for q in range(4):
                            stage[pl.ds(slot * 64 + q * 16, 16)] = (
                                plsc.load_gather(
                                    slabs.at[par], [dvecs[q], cv]
                                )
                            )
                        pltpu.async_copy(
                            stage.at[pl.ds(slot * 64, 64)],
                            out_hbm.at[pl.ds(pos * 64, 64)],
                            sem_w,
                        )

                    nmask = mask_i & jnp.where(liota != jv, 1, 0)
                    return nmask, ms_i + jnp.where(col < SLABW, 1, 0)

                _, ms_out = lax.fori_loop(
                    0, cnt, pop_one, (jnp.where(m, 1, 0), ms)
                )
                return ms_out

            def rescan_group(g, ms):
                for u in range(UNROLL):
                    ms = rescan_one(g * UNROLL + u, ms)
                return ms

            nv = lax.shift_right_logical(nlist + 16 * UNROLL - 1, 6)
            return lax.fori_loop(0, nv, rescan_group, ms0)

        def per_triple(tp, ms):
            for i in range(3):
                st = tp * 3 + i
                ms = process_slab(st, i, ms)

                @pl.when(st + 3 < NSLAB)
                def _():
                    start_slab(st + 3, i)

            return ms

        ms = lax.fori_loop(0, NSLAB // 3, per_triple, jnp.int32(0))
        ms = process_slab(jnp.int32(NSLAB - 1), 0, ms)
        drain_n(jnp.where(ms == 0, 0, ((ms - 1) & (NSTAGE - 1)) + 1))

    return lookup


_lookup = _make_lookup()


def kernel(X, table):
    flat = _lookup(X.astype(jnp.int32), table.T)
    return flat.reshape(BATCH, EMBEDDING_DIM)


# speedup vs baseline: 1.0046x; 1.0046x over previous
import functools

import jax
import jax.numpy as jnp
from jax import lax
from jax.experimental import pallas as pl
from jax.experimental.pallas import tpu as pltpu
from jax.experimental.pallas import tpu_sc as plsc

NUM_EMBEDDINGS = 1000000
EMBEDDING_DIM = 64
BATCH = 16384
NBLOCKS = 7813
SLABW = 384
NSLAB = 82
MAXBASE = 999552
NSTAGE = 128
SCAN_UNROLL = 8
UNROLL = 4


def _make_lookup():
    info = plsc.get_sparse_core_info()

    mesh = plsc.VectorSubcoreMesh(core_axis_name="c", subcore_axis_name="s")

    @functools.partial(
        pl.kernel,
        mesh=mesh,
        out_type=jax.ShapeDtypeStruct((BATCH * EMBEDDING_DIM,), jnp.float32),
        scratch_types=[
            pltpu.VMEM((BATCH,), jnp.int32),
            pltpu.VMEM((BATCH + 64,), jnp.int32),
            pltpu.VMEM((3, EMBEDDING_DIM, SLABW), jnp.float32),
            pltpu.VMEM((NSTAGE * EMBEDDING_DIM,), jnp.float32),
            pltpu.VMEM((EMBEDDING_DIM, 64), jnp.float32),
            pltpu.VMEM((EMBEDDING_DIM,), jnp.float32),
        ]
        + [pltpu.SemaphoreType.DMA] * 5,
        compiler_params=pltpu.CompilerParams(needs_layout_passes=False),
    )
    def lookup(x_hbm, tt_hbm, out_hbm, xall, lists, slabs, stage,
               fbtail, fbcol, sem_s0, sem_s1, sem_s2, sem_w, sem_fb):
        wid = lax.axis_index("s") * info.num_cores + lax.axis_index("c")
        c0 = (NBLOCKS * wid) >> 5
        liota = lax.iota(jnp.int32, 16)
        dvecs = [liota + 16 * q for q in range(4)]
        ssems = [sem_s0, sem_s1, sem_s2]

        def splat(x):
            return jnp.full((16,), x, dtype=jnp.int32)

        c0v = splat(c0)
        c1v = splat((NBLOCKS * (wid + 1)) >> 5)

        def slab_base(st):
            return jnp.minimum((c0 + 3 * st) * 128, MAXBASE)

        def start_slab(st, par):
            pltpu.async_copy(
                tt_hbm.at[:, pl.ds(pl.multiple_of(slab_base(st), 128), SLABW)],
                slabs.at[par], ssems[par],
            )

        def wait_slab(par):
            pltpu.make_async_copy(
                tt_hbm.at[:, pl.ds(0, SLABW)], slabs.at[par], ssems[par]
            ).wait()

        pltpu.sync_copy(x_hbm, xall)
        for par in range(3):
            start_slab(jnp.int32(par), par)

        def tail_fallback(pos, l):
            pltpu.async_copy(
                tt_hbm.at[:, pl.ds(999936, 64)], fbtail, sem_fb
            ).wait()
            lv = splat(l)
            for q in range(4):
                fbcol[pl.ds(q * 16, 16)] = plsc.load_gather(
                    fbtail, [dvecs[q], lv]
                )
            pltpu.sync_copy(fbcol, out_hbm.at[pl.ds(pos * 64, 64)])

        def scan_group(g, ptr):
            recs = []
            for u in range(SCAN_UNROLL):
                k = g * SCAN_UNROLL + u
                v = xall[pl.ds(k * 16, 16)]
                c = lax.shift_right_logical(v, 7)
                m = (c >= c0v) & (c < c1v)
                entryv = (
                    (k * 16 + liota) | ((v & 127) << 14) | ((c - c0v) << 21)
                )
                cnt = plsc.all_reduce_population_count(m)[0]
                recs.append((m, entryv, cnt))
            for m, entryv, cnt in recs:
                plsc.store_compressed(
                    lists.at[pl.ds(ptr, 16)], entryv, mask=m
                )
                ptr = ptr + cnt
            return ptr

        nlist = lax.fori_loop(
            0, (BATCH // 16) // SCAN_UNROLL, scan_group, jnp.int32(0)
        )
        nlv = splat(nlist)

        def drain_all():
            pltpu.make_async_copy(
                out_hbm.at[pl.ds(0, NSTAGE * 64)], stage, sem_w
            ).wait()

        def drain_n(n):
            def one(_, __):
                pltpu.make_async_copy(
                    out_hbm.at[pl.ds(0, 64)], stage.at[pl.ds(0, 64)], sem_w
                ).wait()
                return 0

            lax.fori_loop(0, n, one, 0)

        def process_slab(st, par, ms0):
            wait_slab(par)
            base = slab_base(st)
            stv = splat(st)
            three = splat(3)

            def rescan_one(k, ms):
                ev = lists[pl.ds(k * 16, 16)]
                valid = (k * 16 + liota) < nlv
                rbv = lax.shift_right_logical(ev, 21) & 255
                m = (rbv // three == stv) & valid
                cnt = plsc.all_reduce_population_count(m)[0]

                def pop_one(_, carry):
                    mask_i, ms_i = carry
                    mask = mask_i != 0
                    j = plsc.all_reduce_ffs(mask)[0]
                    jv = splat(j)
                    e = jnp.sum(jnp.where(liota == jv, ev, 0))
                    pos = e & 16383
                    l = lax.shift_right_logical(e, 14) & 127
                    rb = lax.shift_right_logical(e, 21) & 255
                    col = (c0 + rb) * 128 + l - base

                    @pl.when(col >= SLABW)
                    def _():
                        tail_fallback(pos, l)

                    @pl.when(col < SLABW)
                    def _():
                        slot = ms_i & (NSTAGE - 1)

                        @pl.when((slot == 0) & (ms_i > 0))
                        def _():
                            drain_all()

                        cv = splat(col)
                        for q in range(4):
                            stage[pl.ds(slot * 64 + q * 16, 16)] = (
                                plsc.load_gather(
                                    slabs.at[par], [dvecs[q], cv]
                                )
                            )
                        pltpu.async_copy(
                            stage.at[pl.ds(slot * 64, 64)],
                            out_hbm.at[pl.ds(pos * 64, 64)],
                            sem_w,
                        )

                    nmask = mask_i & jnp.where(liota != jv, 1, 0)
                    return nmask, ms_i + jnp.where(col < SLABW, 1, 0)

                _, ms_out = lax.fori_loop(
                    0, cnt, pop_one, (jnp.where(m, 1, 0), ms)
                )
                return ms_out

            def rescan_group(g, ms):
                for u in range(UNROLL):
                    ms = rescan_one(g * UNROLL + u, ms)
                return ms

            nv = lax.shift_right_logical(nlist + 16 * UNROLL - 1, 6)
            return lax.fori_loop(0, nv, rescan_group, ms0)

        def per_triple(tp, ms):
            for i in range(3):
                st = tp * 3 + i
                ms = process_slab(st, i, ms)

                @pl.when(st + 3 < NSLAB)
                def _():
                    start_slab(st + 3, i)

            return ms

        ms = lax.fori_loop(0, NSLAB // 3, per_triple, jnp.int32(0))
        ms = process_slab(jnp.int32(NSLAB - 1), 0, ms)
        drain_n(jnp.where(ms == 0, 0, ((ms - 1) & (NSTAGE - 1)) + 1))

    return lookup


_lookup = _make_lookup()


def kernel(X, table):
    flat = _lookup(X.astype(jnp.int32), table.T)
    return flat.reshape(BATCH, EMBEDDING_DIM)
